# Initial kernel scaffold; baseline (speedup 1.0000x reference)
import jax
import jax.numpy as jnp
from jax.experimental import pallas as pl


def kernel(xyz, features, params):
    raise NotImplementedError("write your pallas kernel here")


# speedup vs baseline: 5.6375x; 5.6375x over previous
import functools
import jax
import jax.numpy as jnp
from jax import lax
from jax.experimental import pallas as pl
from jax.experimental.pallas import tpu as pltpu
from jax.experimental.pallas import tpu_sc as plsc

_N = 2048
_B = 8
_S = 512
_EPS = 1e-5
_RADII = (0.2, 0.4, 0.8)
_KS = (16, 32, 64)
_D = 128


def _fps_body(xyz_ref, nx_ref, ny_ref, nz_ref):
    x = xyz_ref[0]
    y = xyz_ref[1]
    z = xyz_ref[2]
    iota = lax.broadcasted_iota(jnp.int32, (_B, _N), 1)
    col512 = lax.broadcasted_iota(jnp.int32, (_B, _S), 1)

    def body(i, state):
        distances, far, ax, ay, az = state
        oh = (iota == far)
        cx = jnp.sum(jnp.where(oh, x, 0.0), axis=1, keepdims=True)
        cy = jnp.sum(jnp.where(oh, y, 0.0), axis=1, keepdims=True)
        cz = jnp.sum(jnp.where(oh, z, 0.0), axis=1, keepdims=True)
        ax = jnp.where(col512 == i, cx, ax)
        ay = jnp.where(col512 == i, cy, ay)
        az = jnp.where(col512 == i, cz, az)
        dx = x - cx
        dy = y - cy
        dz = z - cz
        dist = dx * dx + dy * dy + dz * dz
        distances = jnp.minimum(distances, dist)
        maxv = jnp.max(distances, axis=1, keepdims=True)
        far = jnp.min(jnp.where(distances == maxv, iota, _N), axis=1,
                      keepdims=True)
        return distances, far, ax, ay, az

    init = (jnp.full((_B, _N), jnp.inf, dtype=jnp.float32),
            jnp.zeros((_B, 1), dtype=jnp.int32),
            jnp.zeros((_B, _S), dtype=jnp.float32),
            jnp.zeros((_B, _S), dtype=jnp.float32),
            jnp.zeros((_B, _S), dtype=jnp.float32))
    _, _, ax, ay, az = lax.fori_loop(0, _S, body, init)
    nx_ref[...] = ax
    ny_ref[...] = ay
    nz_ref[...] = az


def _run_fps(xyz_t):
    out = jax.ShapeDtypeStruct((_B, _S), jnp.float32)
    return pl.pallas_call(
        _fps_body,
        out_shape=(out, out, out),
    )(xyz_t)


_TS = 256


def _bq_body(xyz_ref, cxyz_ref, idx_ref):
    b = pl.program_id(0)
    x = xyz_ref[0, 0:1, :]
    y = xyz_ref[0, 1:2, :]
    z = xyz_ref[0, 2:3, :]
    cx = cxyz_ref[0, :, 0:1]
    cy = cxyz_ref[0, :, 1:2]
    cz = cxyz_ref[0, :, 2:3]
    dot = cx * x + cy * y + cz * z
    ssrc = cx * cx + cy * cy + cz * cz
    sdst = x * x + y * y + z * z
    sq = jnp.maximum(-2.0 * dot + ssrc + sdst, 0.0)
    iota = lax.broadcasted_iota(jnp.int32, (_TS, _N), 1)
    off = b * _N
    cols = []
    for r, k in zip(_RADII, _KS):
        cand0 = jnp.where(sq <= r * r, iota, _N)
        kcol = lax.broadcasted_iota(jnp.int32, (_TS, k), 1)

        def body(j_i, state, kcol=kcol):
            cand, out = state
            j = jnp.min(cand, axis=1, keepdims=True)
            out = jnp.where(kcol == j_i, j, out)
            cand = jnp.where(cand == j, _N, cand)
            return cand, out

        first = jnp.min(cand0, axis=1, keepdims=True)
        _, got = lax.fori_loop(
            0, k, body, (cand0, jnp.zeros((_TS, k), jnp.int32)))
        got = jnp.where(got == _N, first, got)
        cols.append(got + off)
    idx_ref[0] = jnp.concatenate(cols, axis=1)


def _run_ball_query(xyz_t, cxyz):
    ktot = sum(_KS)
    return pl.pallas_call(
        _bq_body,
        grid=(_B, _S // _TS),
        in_specs=[
            pl.BlockSpec((1, 3, _N), lambda b, t: (b, 0, 0)),
            pl.BlockSpec((1, _TS, 3), lambda b, t: (b, t, 0)),
        ],
        out_specs=pl.BlockSpec((1, _TS, ktot), lambda b, t: (b, t, 0)),
        out_shape=jax.ShapeDtypeStruct((_B, _S, ktot), jnp.int32),
    )(xyz_t, cxyz)


_CHUNK = 512


def _sc_gather(table, idx_flat):
    rows = idx_flat.shape[0]
    info = plsc.get_sparse_core_info()
    nw = info.num_cores * info.num_subcores
    per_w = rows // nw
    n_it = per_w // _CHUNK
    mesh = plsc.VectorSubcoreMesh(core_axis_name="c", subcore_axis_name="s")

    @functools.partial(
        pl.kernel, mesh=mesh,
        out_type=jax.ShapeDtypeStruct((rows, _D), jnp.float32),
        scratch_types=[
            pltpu.VMEM((_CHUNK,), jnp.int32),
            pltpu.VMEM((_CHUNK, _D), jnp.float32),
            pltpu.SemaphoreType.DMA,
        ],
    )
    def k(table_hbm, idx_hbm, out_hbm, idx_v, rows_v, sem):
        wid = lax.axis_index("s") * info.num_cores + lax.axis_index("c")
        base = wid * per_w

        def body(it, _):
            o = base + it * _CHUNK
            pltpu.sync_copy(idx_hbm.at[pl.ds(o, _CHUNK)], idx_v)
            pltpu.async_copy(table_hbm.at[idx_v], rows_v, sem).wait()
            pltpu.sync_copy(rows_v, out_hbm.at[pl.ds(o, _CHUNK)])
            return 0

        lax.fori_loop(0, n_it, body, 0)

    return k(table, idx_flat)


_RT = 512


def _l1_body(x_ref, c_ref, w_ref, wx_ref, y_ref, s_ref, ss_ref, *, kk):
    r = pl.program_id(0)
    y = jnp.dot(x_ref[...], w_ref[...], preferred_element_type=jnp.float32)
    corr = jnp.dot(c_ref[0], wx_ref[...], preferred_element_type=jnp.float32)
    ts = _RT // kk
    o = y.shape[1]
    corr_rows = jnp.broadcast_to(corr[:, None, :], (ts, kk, o)).reshape(_RT, o)
    y = y - corr_rows
    y_ref[...] = y

    @pl.when(r == 0)
    def _():
        s_ref[...] = jnp.zeros_like(s_ref)
        ss_ref[...] = jnp.zeros_like(ss_ref)

    s_ref[...] += jnp.sum(y, axis=0, keepdims=True)
    ss_ref[...] += jnp.sum(y * y, axis=0, keepdims=True)


def _mid_body(x_ref, s_ref, ss_ref, g_ref, b_ref, w_ref,
              y_ref, s2_ref, ss2_ref, *, cnt):
    r = pl.program_id(0)
    mean = s_ref[...] / cnt
    var = ss_ref[...] / cnt - mean * mean
    xh = (x_ref[...] - mean) / jnp.sqrt(var + _EPS) * g_ref[...] + b_ref[...]
    xh = jnp.maximum(xh, 0.0)
    y = jnp.dot(xh, w_ref[...], preferred_element_type=jnp.float32)
    y_ref[...] = y

    @pl.when(r == 0)
    def _():
        s2_ref[...] = jnp.zeros_like(s2_ref)
        ss2_ref[...] = jnp.zeros_like(ss2_ref)

    s2_ref[...] += jnp.sum(y, axis=0, keepdims=True)
    ss2_ref[...] += jnp.sum(y * y, axis=0, keepdims=True)


def _pool_body(x_ref, s_ref, ss_ref, g_ref, b_ref, out_ref, *, cnt, kk):
    mean = s_ref[...] / cnt
    var = ss_ref[...] / cnt - mean * mean
    xh = (x_ref[...] - mean) / jnp.sqrt(var + _EPS) * g_ref[...] + b_ref[...]
    xh = jnp.maximum(xh, 0.0)
    ts = _RT // kk
    o = xh.shape[1]
    out_ref[...] = jnp.max(xh.reshape(ts, kk, o), axis=1)


def _stat_spec(o):
    return pl.BlockSpec((1, o), lambda r: (0, 0))


def _run_mlp_scale(grows, cxyz, layers, kk):
    rows = grows.shape[0]
    grid = rows // _RT
    ts = _RT // kk
    lpb = rows // _B // _RT

    (w1, g1, b1), (w2, g2, b2), (w3, g3, b3) = layers
    o1, o2, o3 = w1.shape[0], w2.shape[0], w3.shape[0]
    w1p = jnp.zeros((_D, o1), jnp.float32)
    w1p = w1p.at[0:64, :].set(w1[:, 3:67].T)
    w1p = w1p.at[64:67, :].set(w1[:, 0:3].T)
    wx = w1[:, 0:3].T

    def cspec():
        return pl.BlockSpec(
            (1, ts, 3), lambda r: (r // lpb, r % lpb, 0))

    y1, s1, ss1 = pl.pallas_call(
        functools.partial(_l1_body, kk=kk),
        grid=(grid,),
        in_specs=[
            pl.BlockSpec((_RT, _D), lambda r: (r, 0)),
            cspec(),
            pl.BlockSpec((_D, o1), lambda r: (0, 0)),
            pl.BlockSpec((3, o1), lambda r: (0, 0)),
        ],
        out_specs=(
            pl.BlockSpec((_RT, o1), lambda r: (r, 0)),
            _stat_spec(o1), _stat_spec(o1),
        ),
        out_shape=(
            jax.ShapeDtypeStruct((rows, o1), jnp.float32),
            jax.ShapeDtypeStruct((1, o1), jnp.float32),
            jax.ShapeDtypeStruct((1, o1), jnp.float32),
        ),
    )(grows, cxyz, w1p, wx)

    def mid(x, s, ss, g, b, w, oi, oo):
        return pl.pallas_call(
            functools.partial(_mid_body, cnt=float(rows)),
            grid=(grid,),
            in_specs=[
                pl.BlockSpec((_RT, oi), lambda r: (r, 0)),
                _stat_spec(oi), _stat_spec(oi),
                _stat_spec(oi), _stat_spec(oi),
                pl.BlockSpec((oi, oo), lambda r: (0, 0)),
            ],
            out_specs=(
                pl.BlockSpec((_RT, oo), lambda r: (r, 0)),
                _stat_spec(oo), _stat_spec(oo),
            ),
            out_shape=(
                jax.ShapeDtypeStruct((rows, oo), jnp.float32),
                jax.ShapeDtypeStruct((1, oo), jnp.float32),
                jax.ShapeDtypeStruct((1, oo), jnp.float32),
            ),
        )(x, s, ss, g[None, :], b[None, :], w.T)

    y2, s2, ss2 = mid(y1, s1, ss1, g1, b1, w2, o1, o2)
    y3, s3, ss3 = mid(y2, s2, ss2, g2, b2, w3, o2, o3)

    pooled = pl.pallas_call(
        functools.partial(_pool_body, cnt=float(rows), kk=kk),
        grid=(grid,),
        in_specs=[
            pl.BlockSpec((_RT, o3), lambda r: (r, 0)),
            _stat_spec(o3), _stat_spec(o3),
            _stat_spec(o3), _stat_spec(o3),
        ],
        out_specs=pl.BlockSpec((ts, o3), lambda r: (r, 0)),
        out_shape=jax.ShapeDtypeStruct((rows // kk, o3), jnp.float32),
    )(y3, s3, ss3, g3[None, :], b3[None, :])
    return pooled.reshape(_B, _S, o3)


def kernel(xyz, features, params):
    xyz_t = jnp.transpose(xyz, (2, 0, 1))
    nx, ny, nz = _run_fps(xyz_t)
    new_xyz = jnp.stack([nx, ny, nz], axis=-1)

    idx_all = _run_ball_query(jnp.transpose(xyz, (0, 2, 1)), new_xyz)

    table = jnp.concatenate(
        [features, xyz, jnp.zeros((_B, _N, _D - 67), jnp.float32)],
        axis=-1).reshape(_B * _N, _D)
    offs = [0, _KS[0], _KS[0] + _KS[1], sum(_KS)]
    idx_flat = jnp.concatenate(
        [idx_all[:, :, offs[i]:offs[i + 1]].reshape(-1) for i in range(3)])
    grows = _sc_gather(table, idx_flat)

    feats = []
    row_off = 0
    for i, kk in enumerate(_KS):
        n_rows = _B * _S * kk
        feats.append(_run_mlp_scale(
            grows[row_off:row_off + n_rows], new_xyz, params[i], kk))
        row_off += n_rows
    return new_xyz, jnp.concatenate(feats, axis=-1)
